# baseline (speedup 1.0000x reference)
import functools

import jax
import jax.numpy as jnp
from jax import lax
from jax.experimental import pallas as pl
from jax.experimental.pallas import tpu as pltpu
from jax.experimental.pallas import tpu_sc as plsc

N = 10000
D = 128
H = 16
C = 64

NP = 10240
TILES = 32
CHUNK = 128
ROWS_PER_TILE = NP // TILES
BM = 1024


def _sc_mesh():
    return plsc.VectorSubcoreMesh(core_axis_name="c", subcore_axis_name="s")


_SC_PARAMS = pltpu.CompilerParams(use_tc_tiling_on_sc=False)


def _make_deg_kernel(n_chunks):
    @functools.partial(
        pl.kernel,
        mesh=_sc_mesh(),
        out_type=jax.ShapeDtypeStruct((2, NP, H), jnp.float32),
        scratch_types=[
            pltpu.VMEM((n_chunks, CHUNK), jnp.int32),
            pltpu.VMEM((CHUNK, H), jnp.float32),
            pltpu.SemaphoreType.DMA,
            pltpu.VMEM_SHARED((NP, H), jnp.float32),
        ],
        compiler_params=_SC_PARAMS,
    )
    def deg_kernel(dst_hbm, zeros_hbm, ones_hbm, out_hbm, idx_v, ones_v,
                   ssem, acc_sh):
        c = lax.axis_index("c")
        s = lax.axis_index("s")
        tg = c * 16 + s
        row0 = s * ROWS_PER_TILE
        pltpu.sync_copy(dst_hbm.at[tg], idx_v)
        pltpu.sync_copy(ones_hbm, ones_v)
        pltpu.sync_copy(zeros_hbm, acc_sh.at[pl.ds(row0, ROWS_PER_TILE)])
        plsc.subcore_barrier()

        @pl.loop(0, n_chunks)
        def _(j):
            pltpu.sync_copy(ones_v, acc_sh.at[idx_v.at[j]], add=True)

        plsc.subcore_barrier()
        pltpu.sync_copy(
            acc_sh.at[pl.ds(row0, ROWS_PER_TILE)],
            out_hbm.at[c].at[pl.ds(row0, ROWS_PER_TILE)],
        )

    return deg_kernel


def _make_msg_kernel(n_chunks):
    @functools.partial(
        pl.kernel,
        mesh=_sc_mesh(),
        out_type=jax.ShapeDtypeStruct((2, NP, H), jnp.float32),
        scratch_types=[
            pltpu.VMEM((n_chunks, CHUNK), jnp.int32),
            pltpu.VMEM((n_chunks, CHUNK), jnp.int32),
            pltpu.VMEM((CHUNK, H), jnp.float32),
            pltpu.VMEM((CHUNK, H), jnp.float32),
            pltpu.SemaphoreType.DMA,
            pltpu.SemaphoreType.DMA,
            pltpu.VMEM_SHARED((NP, H), jnp.float32),
        ],
        compiler_params=_SC_PARAMS,
    )
    def msg_kernel(table_hbm, src_hbm, dst_hbm, zeros_hbm, out_hbm,
                   src_v, dst_v, rows_a, rows_b, sem_a, sem_b, acc_sh):
        c = lax.axis_index("c")
        s = lax.axis_index("s")
        tg = c * 16 + s
        row0 = s * ROWS_PER_TILE
        pltpu.sync_copy(src_hbm.at[tg], src_v)
        pltpu.sync_copy(dst_hbm.at[tg], dst_v)
        pltpu.sync_copy(zeros_hbm, acc_sh.at[pl.ds(row0, ROWS_PER_TILE)])
        plsc.subcore_barrier()

        @pl.loop(0, n_chunks, step=2)
        def _(j):
            cpa = pltpu.async_copy(table_hbm.at[src_v.at[j]], rows_a, sem_a)
            cpb = pltpu.async_copy(table_hbm.at[src_v.at[j + 1]], rows_b,
                                   sem_b)
            cpa.wait()
            pltpu.sync_copy(rows_a, acc_sh.at[dst_v.at[j]], add=True)
            cpb.wait()
            pltpu.sync_copy(rows_b, acc_sh.at[dst_v.at[j + 1]], add=True)

        plsc.subcore_barrier()
        pltpu.sync_copy(
            acc_sh.at[pl.ds(row0, ROWS_PER_TILE)],
            out_hbm.at[c].at[pl.ds(row0, ROWS_PER_TILE)],
        )

    return msg_kernel


def _mm1_body(x_ref, w_ref, o_ref):
    o_ref[...] = jnp.dot(x_ref[...], w_ref[...],
                         preferred_element_type=jnp.float32)


def _scale_body(dp_ref, h_ref, o_ref):
    dis = lax.rsqrt(dp_ref[0] + dp_ref[1] + 1.0)
    o_ref[...] = dis * h_ref[...]


def _layer_body(dp_ref, p_ref, hp_ref, w_ref, b_ref, o_ref):
    dis = lax.rsqrt(dp_ref[0] + dp_ref[1] + 1.0)
    z = dis * (p_ref[0] + p_ref[1] + hp_ref[...]) + b_ref[...]
    a = jnp.maximum(z, 0.0)
    h2 = jnp.dot(a, w_ref[...], preferred_element_type=jnp.float32)
    o_ref[...] = dis * h2


def _out_body(dp_ref, q_ref, hp_ref, w_ref, b_ref, bo_ref, o_ref):
    dis = lax.rsqrt(dp_ref[0] + dp_ref[1] + 1.0)
    z = dis * (q_ref[0] + q_ref[1] + hp_ref[...]) + b_ref[...]
    o_ref[...] = jnp.dot(z, w_ref[...],
                         preferred_element_type=jnp.float32) + bo_ref[...]


def _blk(shape, imap):
    return pl.BlockSpec(shape, imap)


def kernel(x, edge_index, W1, b1, W2, b2, Wout, bout):
    E = edge_index.shape[1]
    cap = TILES * CHUNK
    n_chunks = 2 * (-(-E // (2 * cap)))
    e_pad = n_chunks * cap - E
    src = edge_index[0].astype(jnp.int32)
    dst = edge_index[1].astype(jnp.int32)
    if e_pad:
        fill = jnp.arange(e_pad, dtype=jnp.int32)
        src = jnp.concatenate([src, fill % N])
        dst = jnp.concatenate([dst, N + fill % (NP - N)])
    src_t = src.reshape(TILES, n_chunks, CHUNK)
    dst_t = dst.reshape(TILES, n_chunks, CHUNK)
    x_pad = jnp.pad(x, ((0, NP - N), (0, 0)))
    zeros_hbm = jnp.zeros((ROWS_PER_TILE, H), jnp.float32)
    ones_hbm = jnp.ones((CHUNK, H), jnp.float32)
    b1r = b1.reshape(1, H)
    b2r = b2.reshape(1, H)
    boutr = bout.reshape(1, C)

    grid = (NP // BM,)

    h1 = pl.pallas_call(
        _mm1_body,
        grid=grid,
        in_specs=[_blk((BM, D), lambda i: (i, 0)),
                  _blk((D, H), lambda i: (0, 0))],
        out_specs=_blk((BM, H), lambda i: (i, 0)),
        out_shape=jax.ShapeDtypeStruct((NP, H), jnp.float32),
    )(x_pad, W1)

    degp = _make_deg_kernel(n_chunks)(dst_t, zeros_hbm, ones_hbm)

    h1p = pl.pallas_call(
        _scale_body,
        grid=grid,
        in_specs=[_blk((2, BM, H), lambda i: (0, i, 0)),
                  _blk((BM, H), lambda i: (i, 0))],
        out_specs=_blk((BM, H), lambda i: (i, 0)),
        out_shape=jax.ShapeDtypeStruct((NP, H), jnp.float32),
    )(degp, h1)

    p = _make_msg_kernel(n_chunks)(h1p, src_t, dst_t, zeros_hbm)

    h2p = pl.pallas_call(
        _layer_body,
        grid=grid,
        in_specs=[_blk((2, BM, H), lambda i: (0, i, 0)),
                  _blk((2, BM, H), lambda i: (0, i, 0)),
                  _blk((BM, H), lambda i: (i, 0)),
                  _blk((H, H), lambda i: (0, 0)),
                  _blk((1, H), lambda i: (0, 0))],
        out_specs=_blk((BM, H), lambda i: (i, 0)),
        out_shape=jax.ShapeDtypeStruct((NP, H), jnp.float32),
    )(degp, p, h1p, W2, b1r)

    q = _make_msg_kernel(n_chunks)(h2p, src_t, dst_t, zeros_hbm)

    out = pl.pallas_call(
        _out_body,
        grid=grid,
        in_specs=[_blk((2, BM, H), lambda i: (0, i, 0)),
                  _blk((2, BM, H), lambda i: (0, i, 0)),
                  _blk((BM, H), lambda i: (i, 0)),
                  _blk((H, C), lambda i: (0, 0)),
                  _blk((1, H), lambda i: (0, 0)),
                  _blk((1, C), lambda i: (0, 0))],
        out_specs=_blk((BM, C), lambda i: (i, 0)),
        out_shape=jax.ShapeDtypeStruct((NP, C), jnp.float32),
    )(degp, q, h2p, Wout, b2r, boutr)

    return out[:N]


# speedup vs baseline: 1.1400x; 1.1400x over previous
import functools

import jax
import jax.numpy as jnp
from jax import lax
from jax.experimental import pallas as pl
from jax.experimental.pallas import tpu as pltpu
from jax.experimental.pallas import tpu_sc as plsc

N = 10000
D = 128
H = 16
C = 64

NP = 10240
TILES = 32
CHUNK = 128
ROWS_PER_TILE = NP // TILES
BM = 1024


def _sc_mesh():
    return plsc.VectorSubcoreMesh(core_axis_name="c", subcore_axis_name="s")


_SC_PARAMS = pltpu.CompilerParams(use_tc_tiling_on_sc=False)


def _make_deg_kernel(n_chunks):
    @functools.partial(
        pl.kernel,
        mesh=_sc_mesh(),
        out_type=jax.ShapeDtypeStruct((2, NP, H), jnp.float32),
        scratch_types=[
            pltpu.VMEM((n_chunks, CHUNK), jnp.int32),
            pltpu.VMEM((CHUNK, H), jnp.float32),
            pltpu.SemaphoreType.DMA,
            pltpu.VMEM_SHARED((NP, H), jnp.float32),
        ],
        compiler_params=_SC_PARAMS,
    )
    def deg_kernel(dst_hbm, zeros_hbm, ones_hbm, out_hbm, idx_v, ones_v,
                   ssem, acc_sh):
        c = lax.axis_index("c")
        s = lax.axis_index("s")
        tg = c * 16 + s
        row0 = s * ROWS_PER_TILE
        pltpu.sync_copy(dst_hbm.at[tg], idx_v)
        pltpu.sync_copy(ones_hbm, ones_v)
        pltpu.sync_copy(zeros_hbm, acc_sh.at[pl.ds(row0, ROWS_PER_TILE)])
        plsc.subcore_barrier()

        @pl.loop(0, n_chunks, step=8)
        def _(j):
            cps = [
                pltpu.async_copy(ones_v, acc_sh.at[idx_v.at[j + b]], ssem,
                                 add=True)
                for b in range(8)
            ]
            for cp in cps:
                cp.wait()

        plsc.subcore_barrier()
        pltpu.sync_copy(
            acc_sh.at[pl.ds(row0, ROWS_PER_TILE)],
            out_hbm.at[c].at[pl.ds(row0, ROWS_PER_TILE)],
        )

    return deg_kernel


def _make_msg_kernel(n_chunks):
    @functools.partial(
        pl.kernel,
        mesh=_sc_mesh(),
        out_type=jax.ShapeDtypeStruct((2, NP, H), jnp.float32),
        scratch_types=[
            pltpu.VMEM((n_chunks, CHUNK), jnp.int32),
            pltpu.VMEM((n_chunks, CHUNK), jnp.int32),
            pltpu.VMEM((4, CHUNK, H), jnp.float32),
            pltpu.SemaphoreType.DMA,
            pltpu.SemaphoreType.DMA,
            pltpu.SemaphoreType.DMA,
            pltpu.SemaphoreType.DMA,
            pltpu.VMEM_SHARED((NP, H), jnp.float32),
        ],
        compiler_params=_SC_PARAMS,
    )
    def msg_kernel(table_hbm, src_hbm, dst_hbm, zeros_hbm, out_hbm,
                   src_v, dst_v, rows_v, s0, s1, s2, s3, acc_sh):
        c = lax.axis_index("c")
        s = lax.axis_index("s")
        tg = c * 16 + s
        row0 = s * ROWS_PER_TILE
        pltpu.sync_copy(src_hbm.at[tg], src_v)
        pltpu.sync_copy(dst_hbm.at[tg], dst_v)
        pltpu.sync_copy(zeros_hbm, acc_sh.at[pl.ds(row0, ROWS_PER_TILE)])
        plsc.subcore_barrier()

        sems = (s0, s1, s2, s3)

        @pl.loop(0, n_chunks, step=4)
        def _(j):
            cps = [
                pltpu.async_copy(table_hbm.at[src_v.at[j + b]],
                                 rows_v.at[b], sems[b])
                for b in range(4)
            ]
            for b in range(4):
                cps[b].wait()
                pltpu.sync_copy(rows_v.at[b], acc_sh.at[dst_v.at[j + b]],
                                add=True)

        plsc.subcore_barrier()
        pltpu.sync_copy(
            acc_sh.at[pl.ds(row0, ROWS_PER_TILE)],
            out_hbm.at[c].at[pl.ds(row0, ROWS_PER_TILE)],
        )

    return msg_kernel


def _mm1_body(x_ref, w_ref, o_ref):
    o_ref[...] = jnp.dot(x_ref[...], w_ref[...],
                         preferred_element_type=jnp.float32)


def _scale_body(dp_ref, h_ref, o_ref):
    dis = lax.rsqrt(dp_ref[0] + dp_ref[1] + 1.0)
    o_ref[...] = dis * h_ref[...]


def _layer_body(dp_ref, p_ref, hp_ref, w_ref, b_ref, o_ref):
    dis = lax.rsqrt(dp_ref[0] + dp_ref[1] + 1.0)
    z = dis * (p_ref[0] + p_ref[1] + hp_ref[...]) + b_ref[...]
    a = jnp.maximum(z, 0.0)
    h2 = jnp.dot(a, w_ref[...], preferred_element_type=jnp.float32)
    o_ref[...] = dis * h2


def _out_body(dp_ref, q_ref, hp_ref, w_ref, b_ref, bo_ref, o_ref):
    dis = lax.rsqrt(dp_ref[0] + dp_ref[1] + 1.0)
    z = dis * (q_ref[0] + q_ref[1] + hp_ref[...]) + b_ref[...]
    o_ref[...] = jnp.dot(z, w_ref[...],
                         preferred_element_type=jnp.float32) + bo_ref[...]


def _blk(shape, imap):
    return pl.BlockSpec(shape, imap)


def kernel(x, edge_index, W1, b1, W2, b2, Wout, bout):
    E = edge_index.shape[1]
    cap = TILES * CHUNK
    n_chunks = 8 * (-(-E // (8 * cap)))
    e_pad = n_chunks * cap - E
    src = edge_index[0].astype(jnp.int32)
    dst = edge_index[1].astype(jnp.int32)
    if e_pad:
        fill = jnp.arange(e_pad, dtype=jnp.int32)
        src = jnp.concatenate([src, fill % N])
        dst = jnp.concatenate([dst, N + fill % (NP - N)])
    src_t = src.reshape(TILES, n_chunks, CHUNK)
    dst_t = dst.reshape(TILES, n_chunks, CHUNK)
    x_pad = jnp.pad(x, ((0, NP - N), (0, 0)))
    zeros_hbm = jnp.zeros((ROWS_PER_TILE, H), jnp.float32)
    ones_hbm = jnp.ones((CHUNK, H), jnp.float32)
    b1r = b1.reshape(1, H)
    b2r = b2.reshape(1, H)
    boutr = bout.reshape(1, C)

    grid = (NP // BM,)

    h1 = pl.pallas_call(
        _mm1_body,
        grid=grid,
        in_specs=[_blk((BM, D), lambda i: (i, 0)),
                  _blk((D, H), lambda i: (0, 0))],
        out_specs=_blk((BM, H), lambda i: (i, 0)),
        out_shape=jax.ShapeDtypeStruct((NP, H), jnp.float32),
    )(x_pad, W1)

    degp = _make_deg_kernel(n_chunks)(dst_t, zeros_hbm, ones_hbm)

    h1p = pl.pallas_call(
        _scale_body,
        grid=grid,
        in_specs=[_blk((2, BM, H), lambda i: (0, i, 0)),
                  _blk((BM, H), lambda i: (i, 0))],
        out_specs=_blk((BM, H), lambda i: (i, 0)),
        out_shape=jax.ShapeDtypeStruct((NP, H), jnp.float32),
    )(degp, h1)

    p = _make_msg_kernel(n_chunks)(h1p, src_t, dst_t, zeros_hbm)

    h2p = pl.pallas_call(
        _layer_body,
        grid=grid,
        in_specs=[_blk((2, BM, H), lambda i: (0, i, 0)),
                  _blk((2, BM, H), lambda i: (0, i, 0)),
                  _blk((BM, H), lambda i: (i, 0)),
                  _blk((H, H), lambda i: (0, 0)),
                  _blk((1, H), lambda i: (0, 0))],
        out_specs=_blk((BM, H), lambda i: (i, 0)),
        out_shape=jax.ShapeDtypeStruct((NP, H), jnp.float32),
    )(degp, p, h1p, W2, b1r)

    q = _make_msg_kernel(n_chunks)(h2p, src_t, dst_t, zeros_hbm)

    out = pl.pallas_call(
        _out_body,
        grid=grid,
        in_specs=[_blk((2, BM, H), lambda i: (0, i, 0)),
                  _blk((2, BM, H), lambda i: (0, i, 0)),
                  _blk((BM, H), lambda i: (i, 0)),
                  _blk((H, C), lambda i: (0, 0)),
                  _blk((1, H), lambda i: (0, 0)),
                  _blk((1, C), lambda i: (0, 0))],
        out_specs=_blk((BM, C), lambda i: (i, 0)),
        out_shape=jax.ShapeDtypeStruct((NP, C), jnp.float32),
    )(degp, q, h2p, Wout, b2r, boutr)

    return out[:N]
